# baseline (speedup 1.0000x reference)
import jax
import jax.numpy as jnp
from jax import lax
from jax.experimental import pallas as pl
from jax.experimental.pallas import tpu as pltpu
from jax.experimental.pallas import tpu_sc as plsc

NC = 2
NS = 16
NW = NC * NS
L = 16
GRP = 512
CHUNK = 1024
KG = CHUNK // GRP


def _ceil_to(v, m):
    return (v + m - 1) // m * m


def _make_sc_pass(*, n_rows, e_pad, n4p, dwin, dwout, dx, kw, relu_out,
                  want_ew_out, pool_count):
    npool = (2 + dwout) if pool_count else dwout
    psz = 129 * npool * L
    t_pad = e_pad // NW
    n_chunks = t_pad // CHUNK
    grp_per_tile = t_pad // GRP
    zrows = n_rows // NS

    mesh = plsc.VectorSubcoreMesh(core_axis_name="c", subcore_axis_name="s",
                                  num_cores=NC, num_subcores=NS)

    out_type = [
        jax.ShapeDtypeStruct((NC, dx, n_rows), jnp.float32),
        jax.ShapeDtypeStruct((NW, psz), jnp.float32),
    ]
    if want_ew_out:
        out_type.append(jax.ShapeDtypeStruct((dwout, e_pad), jnp.float32))

    scratch = dict(
        bp_b=pltpu.VMEM((n4p,), jnp.int32),
        wb_b=pltpu.VMEM((kw, L), jnp.float32),
        pool_b=pltpu.VMEM((psz,), jnp.float32),
    )
    for b in (0, 1):
        scratch[f"src_b{b}"] = pltpu.VMEM((KG, GRP), jnp.int32)
        scratch[f"dst_b{b}"] = pltpu.VMEM((KG, GRP), jnp.int32)
        scratch[f"ewin_b{b}"] = pltpu.VMEM((dwin, CHUNK), jnp.float32)
        for j in range(dx):
            scratch[f"xs{j}_{b}"] = pltpu.VMEM((CHUNK,), jnp.float32)
            scratch[f"xd{j}_{b}"] = pltpu.VMEM((CHUNK,), jnp.float32)
            scratch[f"msg{j}_{b}"] = pltpu.VMEM((CHUNK,), jnp.float32)
        if want_ew_out:
            scratch[f"ewout_b{b}"] = pltpu.VMEM((dwout, CHUNK), jnp.float32)
        scratch[f"isem{b}"] = pltpu.SemaphoreType.DMA
        scratch[f"gsem{b}"] = pltpu.SemaphoreType.DMA
        scratch[f"ssem{b}"] = pltpu.SemaphoreType.DMA
        scratch[f"esem{b}"] = pltpu.SemaphoreType.DMA
    for j in range(dx):
        scratch[f"agg_spm{j}"] = pltpu.VMEM_SHARED((n_rows,), jnp.float32)

    def body(src_hbm, dst_hbm, ewin_hbm, xcols, bp_hbm, wb_hbm, zeros_hbm,
             agg_hbm, pool_hbm, ewout_hbm, scr):
        c = lax.axis_index("c")
        s = lax.axis_index("s")
        wid = s * NC + c
        bufs = []
        for b in (0, 1):
            bufs.append(dict(
                src=scr[f"src_b{b}"], dst=scr[f"dst_b{b}"],
                ewin=scr[f"ewin_b{b}"],
                xs=[scr[f"xs{j}_{b}"] for j in range(dx)],
                xd=[scr[f"xd{j}_{b}"] for j in range(dx)],
                msg=[scr[f"msg{j}_{b}"] for j in range(dx)],
                ewout=scr.get(f"ewout_b{b}"),
                isem=scr[f"isem{b}"], gsem=scr[f"gsem{b}"],
                ssem=scr[f"ssem{b}"], esem=scr[f"esem{b}"],
            ))
        agg_spm = [scr[f"agg_spm{j}"] for j in range(dx)]

        pltpu.sync_copy(bp_hbm, scr["bp_b"])
        pltpu.sync_copy(wb_hbm, scr["wb_b"])

        zero16 = jnp.zeros((L,), jnp.float32)

        def zpool(i, carry):
            scr["pool_b"][pl.ds(i * L, L)] = zero16
            return carry
        lax.fori_loop(0, psz // L, zpool, 0)

        rslice = pl.ds(s * zrows, zrows)
        for j in range(dx):
            pltpu.sync_copy(zeros_hbm, agg_spm[j].at[rslice])
        plsc.subcore_barrier()

        iota = lax.iota(jnp.int32, L)
        wv = [scr["wb_b"][k, :] for k in range(kw)]
        din = 2 * dx + dwin
        bd_off = din * dwout
        wn_off = bd_off + dwout
        bn_off = wn_off + dwout

        base_grp = wid * grp_per_tile

        def issue_idx(ci, bs):
            grp0 = base_grp + ci * KG
            ebase = grp0 * GRP
            pltpu.async_copy(src_hbm.at[pl.ds(grp0, KG)], bs["src"],
                             bs["isem"])
            pltpu.async_copy(dst_hbm.at[pl.ds(grp0, KG)], bs["dst"],
                             bs["isem"])
            for j in range(dwin):
                pltpu.async_copy(ewin_hbm.at[j, pl.ds(ebase, CHUNK)],
                                 bs["ewin"].at[j], bs["isem"])

        def wait_idx(bs):
            pltpu.make_async_copy(src_hbm.at[pl.ds(0, KG)], bs["src"],
                                  bs["isem"]).wait()
            pltpu.make_async_copy(dst_hbm.at[pl.ds(0, KG)], bs["dst"],
                                  bs["isem"]).wait()
            for j in range(dwin):
                pltpu.make_async_copy(ewin_hbm.at[j, pl.ds(0, CHUNK)],
                                      bs["ewin"].at[j], bs["isem"]).wait()

        def issue_gathers(bs):
            for k in range(KG):
                dsl = pl.ds(k * GRP, GRP)
                for j in range(dx):
                    pltpu.async_copy(xcols[j].at[bs["src"].at[k]],
                                     bs["xs"][j].at[dsl], bs["gsem"])
                    pltpu.async_copy(xcols[j].at[bs["dst"].at[k]],
                                     bs["xd"][j].at[dsl], bs["gsem"])

        def wait_gathers(bs):
            for k in range(KG):
                dsl = pl.ds(k * GRP, GRP)
                for j in range(dx):
                    pltpu.make_async_copy(
                        xcols[j].at[bs["src"].at[k]], bs["xs"][j].at[dsl],
                        bs["gsem"]).wait()
                    pltpu.make_async_copy(
                        xcols[j].at[bs["dst"].at[k]], bs["xd"][j].at[dsl],
                        bs["gsem"]).wait()

        def issue_out(ci, bs):
            ebase = (base_grp + ci * KG) * GRP
            for k in range(KG):
                dsl = pl.ds(k * GRP, GRP)
                for j in range(dx):
                    pltpu.async_copy(bs["msg"][j].at[dsl],
                                     agg_spm[j].at[bs["dst"].at[k]],
                                     bs["ssem"], add=True)
            if want_ew_out:
                for j in range(dwout):
                    pltpu.async_copy(bs["ewout"].at[j],
                                     ewout_hbm.at[j, pl.ds(ebase, CHUNK)],
                                     bs["esem"])

        def wait_out(bs):
            for k in range(KG):
                dsl = pl.ds(k * GRP, GRP)
                for j in range(dx):
                    pltpu.make_async_copy(
                        bs["msg"][j].at[dsl],
                        agg_spm[j].at[bs["dst"].at[k]], bs["ssem"]).wait()
            if want_ew_out:
                for j in range(dwout):
                    pltpu.make_async_copy(bs["ewout"].at[j],
                                          ewout_hbm.at[j, pl.ds(0, CHUNK)],
                                          bs["esem"]).wait()

        def compute(bs):
            def step(i, carry2):
                k = i // (GRP // L)
                r = (i % (GRP // L)) * L
                esl = pl.ds(i * L, L)
                srcv = bs["src"][k, pl.ds(r, L)]
                xs = [bs["xs"][j][esl] for j in range(dx)]
                xd = [bs["xd"][j][esl] for j in range(dx)]
                ewi = [bs["ewin"][j, esl] for j in range(dwin)]
                ins = xs + xd + ewi
                outs = []
                for jo in range(dwout):
                    acc = wv[bd_off + jo]
                    for ji in range(din):
                        acc = acc + ins[ji] * wv[ji * dwout + jo]
                    if relu_out:
                        acc = jnp.maximum(acc, 0.0)
                    outs.append(acc)
                ev = wv[bn_off]
                for jo in range(dwout):
                    ev = ev + outs[jo] * wv[wn_off + jo]
                ev = jnp.maximum(ev, 0.0)
                if want_ew_out:
                    for jo in range(dwout):
                        bs["ewout"][jo, esl] = outs[jo]
                for jx in range(dx):
                    bs["msg"][jx][esl] = ev * xs[jx]
                wordv = plsc.load_gather(
                    scr["bp_b"], [lax.shift_right_logical(srcv, 2)])
                sh = (srcv & 3) * 8
                bev = lax.shift_right_logical(wordv, sh) & 0xFF
                pbase = bev * (npool * L) + iota
                pvals = ([jnp.full((L,), 1.0, jnp.float32), ewi[0]] + outs
                         if pool_count else outs)
                for pc in range(npool):
                    plsc.addupdate_scatter(
                        scr["pool_b"], [pbase + pc * L], pvals[pc])
                return carry2
            lax.fori_loop(0, CHUNK // L, step, 0)

        issue_idx(0, bufs[0])
        wait_idx(bufs[0])
        issue_gathers(bufs[0])

        def pair_body(t, carry):
            for b in (0, 1):
                ci = 2 * t + b
                bs, bo = bufs[b], bufs[1 - b]
                wait_gathers(bs)

                @pl.when(ci >= 2)
                def _():
                    wait_out(bs)

                @pl.when(ci + 1 < n_chunks)
                def _():
                    issue_idx(ci + 1, bo)
                compute(bs)
                issue_out(ci, bs)

                @pl.when(ci + 1 < n_chunks)
                def _():
                    wait_idx(bo)
                    issue_gathers(bo)
            return carry
        lax.fori_loop(0, n_chunks // 2, pair_body, 0)
        wait_out(bufs[0])
        wait_out(bufs[1])

        pltpu.sync_copy(scr["pool_b"], pool_hbm.at[wid])
        plsc.subcore_barrier()

        @pl.when(s == 0)
        def _():
            for j in range(dx):
                pltpu.sync_copy(agg_spm[j], agg_hbm.at[c, j])

    names = list(scratch.keys())
    n_out = len(out_type)

    n_in = 6 + dx

    def body_wrap(*args):
        src_a, dst_a, ewin_a = args[0], args[1], args[2]
        xcols = list(args[3:3 + dx])
        bp_a, wb_a, zeros_a = args[3 + dx], args[4 + dx], args[5 + dx]
        outs = list(args[n_in:n_in + n_out])
        if not want_ew_out:
            outs = outs + [None]
        scr = dict(zip(names, args[n_in + n_out:]))
        return body(src_a, dst_a, ewin_a, xcols, bp_a, wb_a, zeros_a,
                    *outs, scr)

    return pl.kernel(body_wrap, out_type=out_type, mesh=mesh,
                     scratch_types=[scratch[n] for n in names],
                     compiler_params=pltpu.CompilerParams(
                         needs_layout_passes=False,
                         use_tc_tiling_on_sc=False))


def _node_update(agg, xprev, wn_p, wr_p, b_p, bn):
    dx = agg.shape[1]
    n_rows = agg.shape[2]
    pxc = xprev.shape[0]
    grid = n_rows // bn

    def body(agg_ref, xp_ref, wn_ref, wr_ref, b_ref, out_ref):
        a = agg_ref[0] + agg_ref[1]
        xn = (lax.dot_general(wn_ref[...], a, (((0,), (0,)), ((), ())),
                              preferred_element_type=jnp.float32)
              + lax.dot_general(wr_ref[...], xp_ref[...],
                                (((0,), (0,)), ((), ())),
                                preferred_element_type=jnp.float32)
              + b_ref[...])
        out_ref[...] = jnp.maximum(xn, 0.0)

    return pl.pallas_call(
        body,
        grid=(grid,),
        in_specs=[
            pl.BlockSpec((2, dx, bn), lambda i: (0, 0, i)),
            pl.BlockSpec((pxc, bn), lambda i: (0, i)),
            pl.BlockSpec((dx, 4), lambda i: (0, 0)),
            pl.BlockSpec((pxc, 4), lambda i: (0, 0)),
            pl.BlockSpec((4, 1), lambda i: (0, 0)),
        ],
        out_specs=pl.BlockSpec((4, bn), lambda i: (0, i)),
        out_shape=jax.ShapeDtypeStruct((4, n_rows), jnp.float32),
    )(agg, xprev, wn_p, wr_p, b_p)


def _final(agg3, x0t, x1t, x2t, batch2d, p1, p2, p3, gvec, w3n_p, w3r_p, b3_p,
           wm, bm_p, bn, num_g):
    n_rows = agg3.shape[2]
    grid = n_rows // bn
    p1s, p2s, p3s = p1.shape, p2.shape, p3.shape

    def body(agg_ref, x0_ref, x1_ref, x2_ref, b_ref, p1_ref, p2_ref, p3_ref,
             g_ref, w3n_ref, w3r_ref, b3_ref, wm_ref, bm_ref, out_ref,
             acc_ref):
        i = pl.program_id(0)

        @pl.when(i == 0)
        def _():
            acc_ref[...] = jnp.zeros_like(acc_ref)

        a = agg_ref[0] + agg_ref[1]
        x3b = jnp.maximum(
            lax.dot_general(w3n_ref[...], a, (((0,), (0,)), ((), ())),
                            preferred_element_type=jnp.float32)
            + lax.dot_general(w3r_ref[...], x2_ref[...],
                              (((0,), (0,)), ((), ())),
                              preferred_element_type=jnp.float32)
            + b3_ref[...], 0.0)
        xct = jnp.concatenate(
            [jnp.ones((1, bn), jnp.float32), x0_ref[...], x1_ref[:3],
             x2_ref[:3], x3b[:5]], axis=0)
        onehot = (lax.broadcasted_iota(jnp.int32, (num_g, 1), 0)
                  == b_ref[...]).astype(jnp.float32)
        acc_ref[...] += lax.dot_general(
            onehot, xct, (((1,), (1,)), ((), ())),
            preferred_element_type=jnp.float32)

        @pl.when(i == grid - 1)
        def _():
            acc = acc_ref[...]
            ncnt = jnp.maximum(acc[:, 0:1], 1.0)
            nmean = acc[:, 1:13] / ncnt
            p1r = jnp.sum(jnp.sum(p1_ref[...], axis=3), axis=0)
            p2r = jnp.sum(jnp.sum(p2_ref[...], axis=3), axis=0)
            p3r = jnp.sum(jnp.sum(p3_ref[...], axis=3), axis=0)
            ecnt = jnp.maximum(p1r[:num_g, 0:1], 1.0)
            emean = jnp.concatenate(
                [p1r[:num_g, 1:4], p2r[:num_g, 0:3], p3r[:num_g, 0:4]],
                axis=1) / ecnt
            h = jnp.concatenate([nmean, emean, g_ref[...]], axis=1)
            logits = jnp.dot(h, wm_ref[...],
                             preferred_element_type=jnp.float32) + bm_ref[...]
            m = jnp.max(logits, axis=1, keepdims=True)
            lse = m + jnp.log(jnp.sum(jnp.exp(logits - m), axis=1,
                                      keepdims=True))
            out_ref[...] = logits - lse

    return pl.pallas_call(
        body,
        grid=(grid,),
        in_specs=[
            pl.BlockSpec((2, 3, bn), lambda i: (0, 0, i)),
            pl.BlockSpec((1, bn), lambda i: (0, i)),
            pl.BlockSpec((4, bn), lambda i: (0, i)),
            pl.BlockSpec((4, bn), lambda i: (0, i)),
            pl.BlockSpec((1, bn), lambda i: (0, i)),
            pl.BlockSpec(p1s, lambda i: (0,) * len(p1s)),
            pl.BlockSpec(p2s, lambda i: (0,) * len(p2s)),
            pl.BlockSpec(p3s, lambda i: (0,) * len(p3s)),
            pl.BlockSpec((num_g, 1), lambda i: (0, 0)),
            pl.BlockSpec((3, 8), lambda i: (0, 0)),
            pl.BlockSpec((4, 8), lambda i: (0, 0)),
            pl.BlockSpec((8, 1), lambda i: (0, 0)),
            pl.BlockSpec((23, 2), lambda i: (0, 0)),
            pl.BlockSpec((1, 2), lambda i: (0, 0)),
        ],
        out_specs=pl.BlockSpec((num_g, 2), lambda i: (0, 0)),
        out_shape=jax.ShapeDtypeStruct((num_g, 2), jnp.float32),
        scratch_shapes=[pltpu.VMEM((num_g, 13), jnp.float32)],
    )(agg3, x0t, x1t, x2t, batch2d, p1, p2, p3, gvec, w3n_p, w3r_p, b3_p,
      wm, bm_p)


def _pad_w(w, shape):
    out = jnp.zeros(shape, jnp.float32)
    return out.at[:w.shape[0], :w.shape[1]].set(w)


def _wbank(wd, bd, wn, bn_):
    flat = jnp.concatenate([wd.ravel(), bd.ravel(), wn.ravel(), bn_.ravel()])
    return jnp.tile(flat[:, None], (1, L))


@jax.jit
def kernel(x, edge_index, edge_attr, g, batch,
           Wd1, bd1, Wn1, bn1, W1n, W1r, b1,
           Wd2, bd2, Wn2, bn2, W2n, W2r, b2,
           Wd3, bd3, Wn3, bn3, W3n, W3r, b3,
           Wm, bm):
    n = x.shape[0]
    e = edge_index.shape[1]
    num_g = g.shape[0]

    e_pad = _ceil_to(e, NW * CHUNK)
    n_rows = _ceil_to(n + 1, NS * 8 * 4)
    bn_blk = None
    for cand in (2048, 4096, 1024, 512, 256, 128, 64, 32, 16, 8):
        if n_rows % cand == 0:
            bn_blk = cand
            break
    assert bn_blk is not None

    src = edge_index[0]
    dst = edge_index[1]
    padn = e_pad - e
    src_p = jnp.concatenate([src, jnp.full((padn,), n, jnp.int32)])
    dst_p = jnp.concatenate([dst, jnp.full((padn,), n, jnp.int32)])
    src2d = src_p.reshape(e_pad // GRP, GRP)
    dst2d = dst_p.reshape(e_pad // GRP, GRP)
    ea2d = jnp.concatenate([edge_attr[:, 0],
                            jnp.zeros((padn,), jnp.float32)]).reshape(1, e_pad)

    n4p = _ceil_to((n + 1 + 3) // 4, 8)
    bext = jnp.concatenate([batch.astype(jnp.int32),
                            jnp.full((n4p * 4 - n,), num_g, jnp.int32)])
    bw = bext.reshape(n4p, 4)
    bpk = bw[:, 0] | (bw[:, 1] << 8) | (bw[:, 2] << 16) | (bw[:, 3] << 24)

    x0t = jnp.zeros((1, n_rows), jnp.float32).at[0, :n].set(x[:, 0])
    zeros_hbm = jnp.zeros((n_rows // NS,), jnp.float32)

    wb1 = _wbank(Wd1, bd1, Wn1, bn1)
    wb2 = _wbank(Wd2, bd2, Wn2, bn2)
    wb3 = _wbank(Wd3, bd3, Wn3, bn3)

    pass1 = _make_sc_pass(n_rows=n_rows, e_pad=e_pad, n4p=n4p, dwin=1,
                          dwout=2, dx=1, kw=wb1.shape[0],
                          relu_out=True, want_ew_out=True, pool_count=True)
    pass2 = _make_sc_pass(n_rows=n_rows, e_pad=e_pad, n4p=n4p, dwin=2,
                          dwout=3, dx=3, kw=wb2.shape[0],
                          relu_out=True, want_ew_out=True, pool_count=False)
    pass3 = _make_sc_pass(n_rows=n_rows, e_pad=e_pad, n4p=n4p, dwin=3,
                          dwout=4, dx=3, kw=wb3.shape[0],
                          relu_out=False, want_ew_out=False, pool_count=False)

    agg1, p1, ew1 = pass1(src2d, dst2d, ea2d, x0t[0], bpk, wb1, zeros_hbm)
    x1t = _node_update(agg1, x0t,
                       _pad_w(W1n, (1, 4)), _pad_w(W1r, (1, 4)),
                       _pad_w(b1[:, None], (4, 1)), bn_blk)
    agg2, p2, ew2 = pass2(src2d, dst2d, ew1, x1t[0], x1t[1], x1t[2],
                          bpk, wb2, zeros_hbm)
    x2t = _node_update(agg2, x1t,
                       _pad_w(W2n, (3, 4)), _pad_w(W2r, (4, 4)),
                       _pad_w(b2[:, None], (4, 1)), bn_blk)
    agg3, p3 = pass3(src2d, dst2d, ew2, x2t[0], x2t[1], x2t[2],
                     bpk, wb3, zeros_hbm)

    batch2d = jnp.full((1, n_rows), num_g, jnp.int32).at[0, :n].set(batch)
    p1_4d = p1.reshape(NW, 129, 4, L)
    p2_4d = p2.reshape(NW, 129, 3, L)
    p3_4d = p3.reshape(NW, 129, 4, L)

    out = _final(agg3, x0t, x1t, x2t, batch2d, p1_4d, p2_4d, p3_4d, g,
                 _pad_w(W3n, (3, 8)), _pad_w(W3r, (4, 8)),
                 _pad_w(b3[:, None], (8, 1)), Wm,
                 _pad_w(bm[None, :], (1, 2)), bn_blk, num_g)
    return out


# speedup vs baseline: 1.9260x; 1.9260x over previous
import jax
import jax.numpy as jnp
from jax import lax
from jax.experimental import pallas as pl
from jax.experimental.pallas import tpu as pltpu
from jax.experimental.pallas import tpu_sc as plsc

NC = 2
NS = 16
NW = NC * NS
L = 16
GRP = 512
CHUNK = 1024
KG = CHUNK // GRP


def _ceil_to(v, m):
    return (v + m - 1) // m * m


def _make_sc_pass(*, n_rows, e_pad, n4p, dwin, dwout, dx, kw, relu_out,
                  want_ew_out, pool_count):
    npool = (2 + dwout) if pool_count else dwout
    psz = 129 * npool * L
    t_pad = e_pad // NW
    n_chunks = t_pad // CHUNK
    grp_per_tile = t_pad // GRP
    zrows = n_rows // NS

    mesh = plsc.VectorSubcoreMesh(core_axis_name="c", subcore_axis_name="s",
                                  num_cores=NC, num_subcores=NS)

    out_type = [
        jax.ShapeDtypeStruct((NC, dx, n_rows), jnp.float32),
        jax.ShapeDtypeStruct((NW, psz), jnp.float32),
    ]
    if want_ew_out:
        out_type.append(jax.ShapeDtypeStruct((dwout, e_pad), jnp.float32))

    scratch = dict(
        bp_b=pltpu.VMEM((n4p,), jnp.int32),
        wb_b=pltpu.VMEM((kw, L), jnp.float32),
        pool_b=pltpu.VMEM((psz,), jnp.float32),
    )
    for b in (0, 1):
        scratch[f"src_b{b}"] = pltpu.VMEM((KG, GRP), jnp.int32)
        scratch[f"dst_b{b}"] = pltpu.VMEM((KG, GRP), jnp.int32)
        scratch[f"ewin_b{b}"] = pltpu.VMEM((dwin, CHUNK), jnp.float32)
        for j in range(dx):
            scratch[f"xs{j}_{b}"] = pltpu.VMEM((CHUNK,), jnp.float32)
            scratch[f"xd{j}_{b}"] = pltpu.VMEM((CHUNK,), jnp.float32)
            scratch[f"msg{j}_{b}"] = pltpu.VMEM((CHUNK,), jnp.float32)
        if want_ew_out:
            scratch[f"ewout_b{b}"] = pltpu.VMEM((dwout, CHUNK), jnp.float32)
        scratch[f"isem{b}"] = pltpu.SemaphoreType.DMA
        scratch[f"gsem{b}"] = pltpu.SemaphoreType.DMA
        scratch[f"ssem{b}"] = pltpu.SemaphoreType.DMA
        scratch[f"esem{b}"] = pltpu.SemaphoreType.DMA
    for j in range(dx):
        scratch[f"xc_spm{j}"] = pltpu.VMEM_SHARED((n_rows,), jnp.float32)
        scratch[f"agg_spm{j}"] = pltpu.VMEM_SHARED((n_rows,), jnp.float32)

    def body(src_hbm, dst_hbm, ewin_hbm, xtab_hbm, bp_hbm, wb_hbm, zeros_hbm,
             agg_hbm, pool_hbm, ewout_hbm, scr):
        c = lax.axis_index("c")
        s = lax.axis_index("s")
        wid = s * NC + c
        bufs = []
        for b in (0, 1):
            bufs.append(dict(
                src=scr[f"src_b{b}"], dst=scr[f"dst_b{b}"],
                ewin=scr[f"ewin_b{b}"],
                xs=[scr[f"xs{j}_{b}"] for j in range(dx)],
                xd=[scr[f"xd{j}_{b}"] for j in range(dx)],
                msg=[scr[f"msg{j}_{b}"] for j in range(dx)],
                ewout=scr.get(f"ewout_b{b}"),
                isem=scr[f"isem{b}"], gsem=scr[f"gsem{b}"],
                ssem=scr[f"ssem{b}"], esem=scr[f"esem{b}"],
            ))
        xc_spm = [scr[f"xc_spm{j}"] for j in range(dx)]
        agg_spm = [scr[f"agg_spm{j}"] for j in range(dx)]

        pltpu.sync_copy(bp_hbm, scr["bp_b"])
        pltpu.sync_copy(wb_hbm, scr["wb_b"])

        zero16 = jnp.zeros((L,), jnp.float32)

        def zpool(i, carry):
            scr["pool_b"][pl.ds(i * L, L)] = zero16
            return carry
        lax.fori_loop(0, psz // L, zpool, 0)

        rslice = pl.ds(s * zrows, zrows)
        for j in range(dx):
            pltpu.sync_copy(xtab_hbm.at[j, rslice], xc_spm[j].at[rslice])
            pltpu.sync_copy(zeros_hbm, agg_spm[j].at[rslice])
        plsc.subcore_barrier()

        iota = lax.iota(jnp.int32, L)
        wv = [scr["wb_b"][k, :] for k in range(kw)]
        din = 2 * dx + dwin
        bd_off = din * dwout
        wn_off = bd_off + dwout
        bn_off = wn_off + dwout

        base_grp = wid * grp_per_tile

        def issue_idx(ci, bs):
            grp0 = base_grp + ci * KG
            ebase = grp0 * GRP
            pltpu.async_copy(src_hbm.at[pl.ds(grp0, KG)], bs["src"],
                             bs["isem"])
            pltpu.async_copy(dst_hbm.at[pl.ds(grp0, KG)], bs["dst"],
                             bs["isem"])
            for j in range(dwin):
                pltpu.async_copy(ewin_hbm.at[j, pl.ds(ebase, CHUNK)],
                                 bs["ewin"].at[j], bs["isem"])

        def wait_idx(bs):
            pltpu.make_async_copy(src_hbm.at[pl.ds(0, KG)], bs["src"],
                                  bs["isem"]).wait()
            pltpu.make_async_copy(dst_hbm.at[pl.ds(0, KG)], bs["dst"],
                                  bs["isem"]).wait()
            for j in range(dwin):
                pltpu.make_async_copy(ewin_hbm.at[j, pl.ds(0, CHUNK)],
                                      bs["ewin"].at[j], bs["isem"]).wait()

        def issue_gathers(bs):
            for k in range(KG):
                dsl = pl.ds(k * GRP, GRP)
                for j in range(dx):
                    pltpu.async_copy(xc_spm[j].at[bs["src"].at[k]],
                                     bs["xs"][j].at[dsl], bs["gsem"])
                    pltpu.async_copy(xc_spm[j].at[bs["dst"].at[k]],
                                     bs["xd"][j].at[dsl], bs["gsem"])

        def wait_gathers(bs):
            for k in range(KG):
                dsl = pl.ds(k * GRP, GRP)
                for j in range(dx):
                    pltpu.make_async_copy(
                        xc_spm[j].at[bs["src"].at[k]], bs["xs"][j].at[dsl],
                        bs["gsem"]).wait()
                    pltpu.make_async_copy(
                        xc_spm[j].at[bs["dst"].at[k]], bs["xd"][j].at[dsl],
                        bs["gsem"]).wait()

        def issue_out(ci, bs):
            ebase = (base_grp + ci * KG) * GRP
            for k in range(KG):
                dsl = pl.ds(k * GRP, GRP)
                for j in range(dx):
                    pltpu.async_copy(bs["msg"][j].at[dsl],
                                     agg_spm[j].at[bs["dst"].at[k]],
                                     bs["ssem"], add=True)
            if want_ew_out:
                for j in range(dwout):
                    pltpu.async_copy(bs["ewout"].at[j],
                                     ewout_hbm.at[j, pl.ds(ebase, CHUNK)],
                                     bs["esem"])

        def wait_out(bs):
            for k in range(KG):
                dsl = pl.ds(k * GRP, GRP)
                for j in range(dx):
                    pltpu.make_async_copy(
                        bs["msg"][j].at[dsl],
                        agg_spm[j].at[bs["dst"].at[k]], bs["ssem"]).wait()
            if want_ew_out:
                for j in range(dwout):
                    pltpu.make_async_copy(bs["ewout"].at[j],
                                          ewout_hbm.at[j, pl.ds(0, CHUNK)],
                                          bs["esem"]).wait()

        def compute(bs):
            def step(i, carry2):
                k = i // (GRP // L)
                r = (i % (GRP // L)) * L
                esl = pl.ds(i * L, L)
                srcv = bs["src"][k, pl.ds(r, L)]
                xs = [bs["xs"][j][esl] for j in range(dx)]
                xd = [bs["xd"][j][esl] for j in range(dx)]
                ewi = [bs["ewin"][j, esl] for j in range(dwin)]
                ins = xs + xd + ewi
                outs = []
                for jo in range(dwout):
                    acc = wv[bd_off + jo]
                    for ji in range(din):
                        acc = acc + ins[ji] * wv[ji * dwout + jo]
                    if relu_out:
                        acc = jnp.maximum(acc, 0.0)
                    outs.append(acc)
                ev = wv[bn_off]
                for jo in range(dwout):
                    ev = ev + outs[jo] * wv[wn_off + jo]
                ev = jnp.maximum(ev, 0.0)
                if want_ew_out:
                    for jo in range(dwout):
                        bs["ewout"][jo, esl] = outs[jo]
                for jx in range(dx):
                    bs["msg"][jx][esl] = ev * xs[jx]
                wordv = plsc.load_gather(
                    scr["bp_b"], [lax.shift_right_logical(srcv, 2)])
                sh = (srcv & 3) * 8
                bev = lax.shift_right_logical(wordv, sh) & 0xFF
                pbase = bev * (npool * L) + iota
                pvals = ([jnp.full((L,), 1.0, jnp.float32), ewi[0]] + outs
                         if pool_count else outs)
                for pc in range(npool):
                    plsc.addupdate_scatter(
                        scr["pool_b"], [pbase + pc * L], pvals[pc])
                return carry2
            lax.fori_loop(0, CHUNK // L, step, 0)

        issue_idx(0, bufs[0])
        wait_idx(bufs[0])
        issue_gathers(bufs[0])

        def pair_body(t, carry):
            for b in (0, 1):
                ci = 2 * t + b
                bs, bo = bufs[b], bufs[1 - b]
                wait_gathers(bs)

                @pl.when(ci >= 2)
                def _():
                    wait_out(bs)

                @pl.when(ci + 1 < n_chunks)
                def _():
                    issue_idx(ci + 1, bo)
                compute(bs)
                issue_out(ci, bs)

                @pl.when(ci + 1 < n_chunks)
                def _():
                    wait_idx(bo)
                    issue_gathers(bo)
            return carry
        lax.fori_loop(0, n_chunks // 2, pair_body, 0)
        wait_out(bufs[0])
        wait_out(bufs[1])

        pltpu.sync_copy(scr["pool_b"], pool_hbm.at[wid])
        plsc.subcore_barrier()

        @pl.when(s == 0)
        def _():
            for j in range(dx):
                pltpu.sync_copy(agg_spm[j], agg_hbm.at[c, j])

    names = list(scratch.keys())
    n_out = len(out_type)

    def body_wrap(*args):
        ins = args[:7]
        outs = list(args[7:7 + n_out])
        if not want_ew_out:
            outs = outs + [None]
        scr = dict(zip(names, args[7 + n_out:]))
        return body(*ins, *outs, scr)

    return pl.kernel(body_wrap, out_type=out_type, mesh=mesh,
                     scratch_types=[scratch[n] for n in names],
                     compiler_params=pltpu.CompilerParams(
                         needs_layout_passes=False,
                         use_tc_tiling_on_sc=False))


def _node_update(agg, xprev, wn_p, wr_p, b_p, bn):
    dx = agg.shape[1]
    n_rows = agg.shape[2]
    pxc = xprev.shape[0]
    grid = n_rows // bn

    def body(agg_ref, xp_ref, wn_ref, wr_ref, b_ref, out_ref):
        a = agg_ref[0] + agg_ref[1]
        xn = (lax.dot_general(wn_ref[...], a, (((0,), (0,)), ((), ())),
                              preferred_element_type=jnp.float32)
              + lax.dot_general(wr_ref[...], xp_ref[...],
                                (((0,), (0,)), ((), ())),
                                preferred_element_type=jnp.float32)
              + b_ref[...])
        out_ref[...] = jnp.maximum(xn, 0.0)

    return pl.pallas_call(
        body,
        grid=(grid,),
        in_specs=[
            pl.BlockSpec((2, dx, bn), lambda i: (0, 0, i)),
            pl.BlockSpec((pxc, bn), lambda i: (0, i)),
            pl.BlockSpec((dx, 4), lambda i: (0, 0)),
            pl.BlockSpec((pxc, 4), lambda i: (0, 0)),
            pl.BlockSpec((4, 1), lambda i: (0, 0)),
        ],
        out_specs=pl.BlockSpec((4, bn), lambda i: (0, i)),
        out_shape=jax.ShapeDtypeStruct((4, n_rows), jnp.float32),
    )(agg, xprev, wn_p, wr_p, b_p)


def _final(agg3, x0t, x1t, x2t, batch2d, p1, p2, p3, gvec, w3n_p, w3r_p, b3_p,
           wm, bm_p, bn, num_g):
    n_rows = agg3.shape[2]
    grid = n_rows // bn
    p1s, p2s, p3s = p1.shape, p2.shape, p3.shape

    def body(agg_ref, x0_ref, x1_ref, x2_ref, b_ref, p1_ref, p2_ref, p3_ref,
             g_ref, w3n_ref, w3r_ref, b3_ref, wm_ref, bm_ref, out_ref,
             acc_ref):
        i = pl.program_id(0)

        @pl.when(i == 0)
        def _():
            acc_ref[...] = jnp.zeros_like(acc_ref)

        a = agg_ref[0] + agg_ref[1]
        x3b = jnp.maximum(
            lax.dot_general(w3n_ref[...], a, (((0,), (0,)), ((), ())),
                            preferred_element_type=jnp.float32)
            + lax.dot_general(w3r_ref[...], x2_ref[...],
                              (((0,), (0,)), ((), ())),
                              preferred_element_type=jnp.float32)
            + b3_ref[...], 0.0)
        xct = jnp.concatenate(
            [jnp.ones((1, bn), jnp.float32), x0_ref[...], x1_ref[:3],
             x2_ref[:3], x3b[:5]], axis=0)
        onehot = (lax.broadcasted_iota(jnp.int32, (num_g, 1), 0)
                  == b_ref[...]).astype(jnp.float32)
        acc_ref[...] += lax.dot_general(
            onehot, xct, (((1,), (1,)), ((), ())),
            preferred_element_type=jnp.float32)

        @pl.when(i == grid - 1)
        def _():
            acc = acc_ref[...]
            ncnt = jnp.maximum(acc[:, 0:1], 1.0)
            nmean = acc[:, 1:13] / ncnt
            p1r = jnp.sum(jnp.sum(p1_ref[...], axis=3), axis=0)
            p2r = jnp.sum(jnp.sum(p2_ref[...], axis=3), axis=0)
            p3r = jnp.sum(jnp.sum(p3_ref[...], axis=3), axis=0)
            ecnt = jnp.maximum(p1r[:num_g, 0:1], 1.0)
            emean = jnp.concatenate(
                [p1r[:num_g, 1:4], p2r[:num_g, 0:3], p3r[:num_g, 0:4]],
                axis=1) / ecnt
            h = jnp.concatenate([nmean, emean, g_ref[...]], axis=1)
            logits = jnp.dot(h, wm_ref[...],
                             preferred_element_type=jnp.float32) + bm_ref[...]
            m = jnp.max(logits, axis=1, keepdims=True)
            lse = m + jnp.log(jnp.sum(jnp.exp(logits - m), axis=1,
                                      keepdims=True))
            out_ref[...] = logits - lse

    return pl.pallas_call(
        body,
        grid=(grid,),
        in_specs=[
            pl.BlockSpec((2, 3, bn), lambda i: (0, 0, i)),
            pl.BlockSpec((1, bn), lambda i: (0, i)),
            pl.BlockSpec((4, bn), lambda i: (0, i)),
            pl.BlockSpec((4, bn), lambda i: (0, i)),
            pl.BlockSpec((1, bn), lambda i: (0, i)),
            pl.BlockSpec(p1s, lambda i: (0,) * len(p1s)),
            pl.BlockSpec(p2s, lambda i: (0,) * len(p2s)),
            pl.BlockSpec(p3s, lambda i: (0,) * len(p3s)),
            pl.BlockSpec((num_g, 1), lambda i: (0, 0)),
            pl.BlockSpec((3, 8), lambda i: (0, 0)),
            pl.BlockSpec((4, 8), lambda i: (0, 0)),
            pl.BlockSpec((8, 1), lambda i: (0, 0)),
            pl.BlockSpec((23, 2), lambda i: (0, 0)),
            pl.BlockSpec((1, 2), lambda i: (0, 0)),
        ],
        out_specs=pl.BlockSpec((num_g, 2), lambda i: (0, 0)),
        out_shape=jax.ShapeDtypeStruct((num_g, 2), jnp.float32),
        scratch_shapes=[pltpu.VMEM((num_g, 13), jnp.float32)],
    )(agg3, x0t, x1t, x2t, batch2d, p1, p2, p3, gvec, w3n_p, w3r_p, b3_p,
      wm, bm_p)


def _pad_w(w, shape):
    out = jnp.zeros(shape, jnp.float32)
    return out.at[:w.shape[0], :w.shape[1]].set(w)


def _wbank(wd, bd, wn, bn_):
    flat = jnp.concatenate([wd.ravel(), bd.ravel(), wn.ravel(), bn_.ravel()])
    return jnp.tile(flat[:, None], (1, L))


@jax.jit
def kernel(x, edge_index, edge_attr, g, batch,
           Wd1, bd1, Wn1, bn1, W1n, W1r, b1,
           Wd2, bd2, Wn2, bn2, W2n, W2r, b2,
           Wd3, bd3, Wn3, bn3, W3n, W3r, b3,
           Wm, bm):
    n = x.shape[0]
    e = edge_index.shape[1]
    num_g = g.shape[0]

    e_pad = _ceil_to(e, NW * CHUNK)
    n_rows = _ceil_to(n + 1, NS * 8 * 4)
    bn_blk = None
    for cand in (2048, 4096, 1024, 512, 256, 128, 64, 32, 16, 8):
        if n_rows % cand == 0:
            bn_blk = cand
            break
    assert bn_blk is not None

    src = edge_index[0]
    dst = edge_index[1]
    padn = e_pad - e
    src_p = jnp.concatenate([src, jnp.full((padn,), n, jnp.int32)])
    dst_p = jnp.concatenate([dst, jnp.full((padn,), n, jnp.int32)])
    src2d = src_p.reshape(e_pad // GRP, GRP)
    dst2d = dst_p.reshape(e_pad // GRP, GRP)
    ea2d = jnp.concatenate([edge_attr[:, 0],
                            jnp.zeros((padn,), jnp.float32)]).reshape(1, e_pad)

    n4p = _ceil_to((n + 1 + 3) // 4, 8)
    bext = jnp.concatenate([batch.astype(jnp.int32),
                            jnp.full((n4p * 4 - n,), num_g, jnp.int32)])
    bw = bext.reshape(n4p, 4)
    bpk = bw[:, 0] | (bw[:, 1] << 8) | (bw[:, 2] << 16) | (bw[:, 3] << 24)

    x0t = jnp.zeros((1, n_rows), jnp.float32).at[0, :n].set(x[:, 0])
    zeros_hbm = jnp.zeros((n_rows // NS,), jnp.float32)

    wb1 = _wbank(Wd1, bd1, Wn1, bn1)
    wb2 = _wbank(Wd2, bd2, Wn2, bn2)
    wb3 = _wbank(Wd3, bd3, Wn3, bn3)

    pass1 = _make_sc_pass(n_rows=n_rows, e_pad=e_pad, n4p=n4p, dwin=1,
                          dwout=2, dx=1, kw=wb1.shape[0],
                          relu_out=True, want_ew_out=True, pool_count=True)
    pass2 = _make_sc_pass(n_rows=n_rows, e_pad=e_pad, n4p=n4p, dwin=2,
                          dwout=3, dx=3, kw=wb2.shape[0],
                          relu_out=True, want_ew_out=True, pool_count=False)
    pass3 = _make_sc_pass(n_rows=n_rows, e_pad=e_pad, n4p=n4p, dwin=3,
                          dwout=4, dx=3, kw=wb3.shape[0],
                          relu_out=False, want_ew_out=False, pool_count=False)

    agg1, p1, ew1 = pass1(src2d, dst2d, ea2d, x0t, bpk, wb1, zeros_hbm)
    x1t = _node_update(agg1, x0t,
                       _pad_w(W1n, (1, 4)), _pad_w(W1r, (1, 4)),
                       _pad_w(b1[:, None], (4, 1)), bn_blk)
    agg2, p2, ew2 = pass2(src2d, dst2d, ew1, x1t, bpk, wb2, zeros_hbm)
    x2t = _node_update(agg2, x1t,
                       _pad_w(W2n, (3, 4)), _pad_w(W2r, (4, 4)),
                       _pad_w(b2[:, None], (4, 1)), bn_blk)
    agg3, p3 = pass3(src2d, dst2d, ew2, x2t, bpk, wb3, zeros_hbm)

    batch2d = jnp.full((1, n_rows), num_g, jnp.int32).at[0, :n].set(batch)
    p1_4d = p1.reshape(NW, 129, 4, L)
    p2_4d = p2.reshape(NW, 129, 3, L)
    p3_4d = p3.reshape(NW, 129, 4, L)

    out = _final(agg3, x0t, x1t, x2t, batch2d, p1_4d, p2_4d, p3_4d, g,
                 _pad_w(W3n, (3, 8)), _pad_w(W3r, (4, 8)),
                 _pad_w(b3[:, None], (8, 1)), Wm,
                 _pad_w(bm[None, :], (1, 2)), bn_blk, num_g)
    return out
